# Initial kernel scaffold; baseline (speedup 1.0000x reference)
import jax
import jax.numpy as jnp
from jax.experimental import pallas as pl


def kernel(warped_xyz, warped_points, RF3, RF3_index, lidar_z, gt_project, c0_W, c0_b, c0_g, c0_be, c1_W, c1_b, c1_g, c1_be, c2_W, c2_b, c2_g, c2_be, m0_W, m0_b, m0_g, m0_be, m1_W, m1_b, m1_g, m1_be, m2_W, m2_b):
    raise NotImplementedError("write your pallas kernel here")


# speedup vs baseline: 14.4983x; 14.4983x over previous
import functools

import jax
import jax.numpy as jnp
from jax import lax
from jax.experimental import pallas as pl
from jax.experimental.pallas import tpu as pltpu
from jax.experimental.pallas import tpu_sc as plsc

F32 = jnp.float32
K = 16
QB = 256
C0, C1, C2 = 128, 64, 64


def _dot(a, b):
    return jnp.dot(a, b, preferred_element_type=F32)


def _knn_body(q_ref, st_ref, sr_ref, out_ref):
    q = q_ref[0]
    st = st_ref[0]
    sr = sr_ref[0]
    M = st.shape[1]
    q2 = jnp.sum(q * q, axis=1, keepdims=True)
    s2 = jnp.sum(st * st, axis=0, keepdims=True)
    e = lax.dot_general(q, sr, (((1,), (1,)), ((), ())),
                        preferred_element_type=F32)
    d = (q2 + s2) - 2.0 * e
    cols = lax.broadcasted_iota(jnp.int32, d.shape, 1)
    base = pl.program_id(0) * M
    picks = []
    for _ in range(K):
        m = jnp.min(d, axis=1, keepdims=True)
        sel = jnp.where(d <= m, cols, jnp.int32(2**30))
        ix = jnp.min(sel, axis=1, keepdims=True)
        picks.append(ix + base)
        d = jnp.where(cols == ix, jnp.float32(jnp.inf), d)
    out_ref[0] = jnp.concatenate(picks, axis=1)


def _knn(q_xyz, st, srows):
    B, N, _ = q_xyz.shape
    M = st.shape[2]
    return pl.pallas_call(
        _knn_body,
        grid=(B, N // QB),
        in_specs=[
            pl.BlockSpec((1, QB, 3), lambda b, i: (b, i, 0)),
            pl.BlockSpec((1, 3, M), lambda b, i: (b, 0, 0)),
            pl.BlockSpec((1, M, 3), lambda b, i: (b, 0, 0)),
        ],
        out_specs=pl.BlockSpec((1, QB, K), lambda b, i: (b, i, 0)),
        out_shape=jax.ShapeDtypeStruct((B, N, K), jnp.int32),
    )(q_xyz, st, srows)


def _sc_gather(table, idx_flat):
    n_idx = idx_flat.shape[0]
    idx2 = idx_flat.reshape(1, n_idx)
    mesh = plsc.VectorSubcoreMesh(core_axis_name="c", subcore_axis_name="s")
    win = 128

    @functools.partial(
        pl.kernel,
        out_type=jax.ShapeDtypeStruct((n_idx, 128), F32),
        mesh=mesh,
    )
    def k(t_hbm, i_hbm, o_hbm):
        def body(i_vmem, o_vmem):
            pltpu.sync_copy(t_hbm.at[i_vmem.at[0]], o_vmem)

        pltpu.emit_pipeline(
            body,
            grid=(n_idx // win,),
            in_specs=[pl.BlockSpec((1, win), lambda i: (0, i))],
            out_specs=[pl.BlockSpec((win, 128), lambda i: (i, 0))],
            core_axis_name=("c", "s"),
            dimension_semantics=(pltpu.PARALLEL,),
        )(i_hbm, o_hbm)

    return k(table, idx2)


def _y0_block(q, g, w0a, w0q, w0g, b0):
    qb = q.shape[0]
    wxyz = q[:, 0:3] * q[:, 3:4]
    y0q = _dot(wxyz, w0a) + _dot(q, w0q) + b0
    g2 = g.reshape(qb * K, g.shape[2])
    y0g = _dot(g2, w0g)
    return y0g + jnp.broadcast_to(y0q[:, None, :], (qb, K, C0)).reshape(qb * K, C0)


def _acc_stats(i, s_ref, ss_ref, y):
    @pl.when(i == 0)
    def _():
        s_ref[...] = jnp.zeros_like(s_ref)
        ss_ref[...] = jnp.zeros_like(ss_ref)

    s_ref[...] += jnp.sum(y, axis=0, keepdims=True)
    ss_ref[...] += jnp.sum(y * y, axis=0, keepdims=True)


def _bn_relu(y, s, ss, gamma, beta, count):
    mean = s / count
    var = jnp.maximum(ss / count - mean * mean, 0.0)
    return jnp.maximum((y - mean) * lax.rsqrt(var + 1e-5) * gamma + beta, 0.0)


def _stats0_body(g_ref, q_ref, w0a_ref, w0q_ref, w0g_ref, b0_ref,
                 s_ref, ss_ref):
    y0 = _y0_block(q_ref[...], g_ref[...], w0a_ref[...], w0q_ref[...],
                   w0g_ref[...], b0_ref[...])
    _acc_stats(pl.program_id(0), s_ref, ss_ref, y0)


def _l1_body(g_ref, q_ref, w0a_ref, w0q_ref, w0g_ref, b0_ref,
             s0_ref, ss0_ref, g0_ref, be0_ref, w1_ref, b1_ref,
             y1_ref, gx_ref, s_ref, ss_ref, *, count):
    gx_ref[...] = g_ref[...][:, :, 64:72]
    y0 = _y0_block(q_ref[...], g_ref[...], w0a_ref[...], w0q_ref[...],
                   w0g_ref[...], b0_ref[...])
    x1 = _bn_relu(y0, s0_ref[...], ss0_ref[...], g0_ref[...], be0_ref[...],
                  count)
    y1 = _dot(x1, w1_ref[...]) + b1_ref[...]
    y1_ref[...] = y1.reshape(QB, K, C1)
    _acc_stats(pl.program_id(0), s_ref, ss_ref, y1)


def _l2_body(y1_ref, s1_ref, ss1_ref, g1_ref, be1_ref, w2_ref, b2_ref,
             y2_ref, s_ref, ss_ref, *, count):
    y1 = y1_ref[...].reshape(QB * K, C1)
    x2 = _bn_relu(y1, s1_ref[...], ss1_ref[...], g1_ref[...], be1_ref[...],
                  count)
    y2 = _dot(x2, w2_ref[...]) + b2_ref[...]
    y2_ref[...] = y2.reshape(QB, K, C2)
    _acc_stats(pl.program_id(0), s_ref, ss_ref, y2)


def _agg_body(y2_ref, gx_ref, s2_ref, ss2_ref, g2_ref, be2_ref,
              af_ref, uv_ref, *, count):
    y2 = y2_ref[...].reshape(QB * K, C2)
    feats = _bn_relu(y2, s2_ref[...], ss2_ref[...], g2_ref[...], be2_ref[...],
                     count).reshape(QB, K, C2)
    mx = jnp.max(feats, axis=2)
    e = jnp.exp(mx - jnp.max(mx, axis=1, keepdims=True))
    aw = e / jnp.sum(e, axis=1, keepdims=True)
    af_ref[...] = jnp.sum(aw[:, :, None] * feats, axis=1)
    gx = gx_ref[...][:, :, 0:3]
    uv_ref[...] = jnp.sum(aw[:, :, None] * gx, axis=1)


def _mask(p, q):
    r = lax.broadcasted_iota(jnp.int32, (3, 3), 0)
    c = lax.broadcasted_iota(jnp.int32, (3, 3), 1)
    return ((r == p) & (c == q)).astype(F32)


def _jacobi_svd_r_t(Hm, cs, ct):
    eye = _mask(0, 0) + _mask(1, 1) + _mask(2, 2)
    A = lax.dot_general(Hm, Hm, (((0,), (0,)), ((), ())))
    V = eye
    for _ in range(6):
        for (p, q) in ((0, 1), (0, 2), (1, 2)):
            app = A[p:p + 1, p:p + 1]
            aqq = A[q:q + 1, q:q + 1]
            apq = A[p:p + 1, q:q + 1]
            nz = jnp.abs(apq) > 1e-30
            apq_s = jnp.where(nz, apq, 1.0)
            tau = (aqq - app) / (2.0 * apq_s)
            sg = jnp.where(tau >= 0.0, 1.0, -1.0)
            t = sg / (jnp.abs(tau) + jnp.sqrt(1.0 + tau * tau))
            t = jnp.where(nz, t, 0.0)
            c = lax.rsqrt(1.0 + t * t)
            s = t * c
            J = eye + (c - 1.0) * (_mask(p, p) + _mask(q, q)) \
                + s * _mask(p, q) - s * _mask(q, p)
            A = _dot(lax.dot_general(J, A, (((0,), (0,)), ((), ()))), J)
            V = _dot(V, J)
    l0 = A[0:1, 0:1]
    l1 = A[1:2, 1:2]
    l2 = A[2:3, 2:3]
    detv = jnp.ones_like(l0)
    def swap(li, lj, V, detv, i, j):
        cnd = li < lj
        P = eye - _mask(i, i) - _mask(j, j) + _mask(i, j) + _mask(j, i)
        Vn = jnp.where(cnd, _dot(V, P), V)
        dn = jnp.where(cnd, -detv, detv)
        return jnp.where(cnd, lj, li), jnp.where(cnd, li, lj), Vn, dn
    l0, l1, V, detv = swap(l0, l1, V, detv, 0, 1)
    l0, l2, V, detv = swap(l0, l2, V, detv, 0, 2)
    l1, l2, V, detv = swap(l1, l2, V, detv, 1, 2)
    HV = _dot(Hm, V)
    norm2 = jnp.sum(HV * HV, axis=0, keepdims=True)
    Un = HV * lax.rsqrt(norm2 + 1e-30)
    a = Un[:, 0:1]
    b = Un[:, 1:2]
    u2 = jnp.concatenate(
        [a[1:2] * b[2:3] - a[2:3] * b[1:2],
         a[2:3] * b[0:1] - a[0:1] * b[2:3],
         a[0:1] * b[1:2] - a[1:2] * b[0:1]], axis=0)
    U = jnp.concatenate([a, b, u2], axis=1)
    d_row = jnp.concatenate([jnp.ones_like(detv), jnp.ones_like(detv), detv],
                            axis=1)
    R = lax.dot_general(V * d_row, U, (((1,), (1,)), ((), ())))
    t = ct - lax.dot_general(cs, R, (((1,), (1,)), ((), ())))
    return R, t


def _head_body(af_ref, uv_ref, xyz_ref, z_ref, gt_ref,
               m0w_ref, m0b_ref, m0g_ref, m0be_ref,
               m1w_ref, m1b_ref, m1g_ref, m1be_ref,
               m2w_ref, m2b_ref,
               r_ref, t_ref, w_ref, *, n_rows, n_pts):
    a = af_ref[...]
    y = _dot(a, m0w_ref[...]) + m0b_ref[...]
    s = jnp.sum(y, axis=0, keepdims=True)
    ss = jnp.sum(y * y, axis=0, keepdims=True)
    x = _bn_relu(y, s, ss, m0g_ref[...], m0be_ref[...], float(n_rows))
    y = _dot(x, m1w_ref[...]) + m1b_ref[...]
    s = jnp.sum(y, axis=0, keepdims=True)
    ss = jnp.sum(y * y, axis=0, keepdims=True)
    x = _bn_relu(y, s, ss, m1g_ref[...], m1be_ref[...], float(n_rows))
    w = _dot(x, m2w_ref[...]) + m2b_ref[...]
    w_ref[...] = w.reshape(w_ref.shape)

    for b in range(w_ref.shape[0]):
        gt = gt_ref[b]
        win = jnp.where(gt[:, 1:2] > gt[:, 0:1], 1.0, 0.0)
        wn = win / (jnp.sum(win) + 1e-8)
        src = xyz_ref[b] * z_ref[b]
        uv = uv_ref[pl.ds(b * n_pts, n_pts), 0:2]
        tgt = jnp.concatenate([uv, jnp.ones_like(uv[:, 0:1])], axis=1)
        cs = jnp.sum(wn * src, axis=0, keepdims=True)
        ct = jnp.sum(wn * tgt, axis=0, keepdims=True)
        sc = src - cs
        tc = tgt - ct
        Hm = lax.dot_general(wn * sc, tc, (((0,), (0,)), ((), ())))
        R, t = _jacobi_svd_r_t(Hm, cs, ct)
        r_ref[b] = R
        t_ref[pl.ds(b, 1), :] = t


def kernel(warped_xyz, warped_points, RF3, RF3_index, lidar_z, gt_project,
           c0_W, c0_b, c0_g, c0_be, c1_W, c1_b, c1_g, c1_be,
           c2_W, c2_b, c2_g, c2_be, m0_W, m0_b, m0_g, m0_be,
           m1_W, m1_b, m1_g, m1_be, m2_W, m2_b):
    B, N, _ = warped_xyz.shape
    _, Ci, H, W_ = RF3.shape
    M = H * W_
    BN = B * N
    rows = BN * K
    count = float(rows)

    st = RF3_index.reshape(B, 3, M)
    srows = RF3_index.transpose(0, 2, 3, 1).reshape(B, M, 3)
    idx = _knn(warped_xyz, st, srows)

    feat_t = RF3.transpose(0, 2, 3, 1).reshape(B * M, Ci)
    xyz_t = srows.reshape(B * M, 3)
    table = jnp.concatenate(
        [feat_t, xyz_t, jnp.zeros((B * M, 61), F32)], axis=1)

    g_rows = _sc_gather(table, idx.reshape(rows))
    g3 = g_rows.reshape(BN, K, 128)

    q_cat = jnp.concatenate(
        [warped_xyz.reshape(BN, 3), lidar_z.reshape(BN, 1),
         warped_points.reshape(BN, Ci)], axis=1)

    w0a = c0_W[0:3]
    w0q = jnp.concatenate([jnp.zeros((4, C0), F32), c0_W[6:70]], axis=0)
    w0g = jnp.concatenate([c0_W[70:134], c0_W[3:6],
                           jnp.zeros((61, C0), F32)], axis=0)

    nblk = BN // QB
    g_spec = pl.BlockSpec((QB, K, 128), lambda i: (i, 0, 0))
    q_spec = pl.BlockSpec((QB, 68), lambda i: (i, 0))
    full = lambda *shape: pl.BlockSpec(shape, lambda i: tuple(0 for _ in shape))
    acc_spec = lambda c: pl.BlockSpec((1, c), lambda i: (0, 0))

    s0, ss0 = pl.pallas_call(
        _stats0_body,
        grid=(nblk,),
        in_specs=[g_spec, q_spec, full(3, C0), full(68, C0), full(128, C0),
                  full(1, C0)],
        out_specs=[acc_spec(C0), acc_spec(C0)],
        out_shape=[jax.ShapeDtypeStruct((1, C0), F32)] * 2,
    )(g3, q_cat, w0a, w0q, w0g, c0_b.reshape(1, C0))

    y1, gx3, s1, ss1 = pl.pallas_call(
        functools.partial(_l1_body, count=count),
        grid=(nblk,),
        in_specs=[g_spec, q_spec, full(3, C0), full(68, C0), full(128, C0),
                  full(1, C0), full(1, C0), full(1, C0), full(1, C0),
                  full(1, C0), full(C0, C1), full(1, C1)],
        out_specs=[pl.BlockSpec((QB, K, C1), lambda i: (i, 0, 0)),
                   pl.BlockSpec((QB, K, 8), lambda i: (i, 0, 0)),
                   acc_spec(C1), acc_spec(C1)],
        out_shape=[jax.ShapeDtypeStruct((BN, K, C1), F32),
                   jax.ShapeDtypeStruct((BN, K, 8), F32),
                   jax.ShapeDtypeStruct((1, C1), F32),
                   jax.ShapeDtypeStruct((1, C1), F32)],
    )(g3, q_cat, w0a, w0q, w0g, c0_b.reshape(1, C0), s0, ss0,
      c0_g.reshape(1, C0), c0_be.reshape(1, C0), c1_W, c1_b.reshape(1, C1))

    y2, s2, ss2 = pl.pallas_call(
        functools.partial(_l2_body, count=count),
        grid=(nblk,),
        in_specs=[pl.BlockSpec((QB, K, C1), lambda i: (i, 0, 0)),
                  full(1, C1), full(1, C1), full(1, C1), full(1, C1),
                  full(C1, C2), full(1, C2)],
        out_specs=[pl.BlockSpec((QB, K, C2), lambda i: (i, 0, 0)),
                   acc_spec(C2), acc_spec(C2)],
        out_shape=[jax.ShapeDtypeStruct((BN, K, C2), F32),
                   jax.ShapeDtypeStruct((1, C2), F32),
                   jax.ShapeDtypeStruct((1, C2), F32)],
    )(y1, s1, ss1, c1_g.reshape(1, C1), c1_be.reshape(1, C1),
      c2_W, c2_b.reshape(1, C2))

    af, uv = pl.pallas_call(
        functools.partial(_agg_body, count=count),
        grid=(nblk,),
        in_specs=[pl.BlockSpec((QB, K, C2), lambda i: (i, 0, 0)),
                  pl.BlockSpec((QB, K, 8), lambda i: (i, 0, 0)),
                  full(1, C2), full(1, C2), full(1, C2), full(1, C2)],
        out_specs=[pl.BlockSpec((QB, C2), lambda i: (i, 0)),
                   pl.BlockSpec((QB, 3), lambda i: (i, 0))],
        out_shape=[jax.ShapeDtypeStruct((BN, C2), F32),
                   jax.ShapeDtypeStruct((BN, 3), F32)],
    )(y2, gx3, s2, ss2, c2_g.reshape(1, C2), c2_be.reshape(1, C2))

    C3 = m1_W.shape[1]
    fullb = lambda *shape: pl.BlockSpec(shape, lambda: tuple(0 for _ in shape))
    R, t, weights = pl.pallas_call(
        functools.partial(_head_body, n_rows=BN, n_pts=N),
        in_specs=[fullb(BN, C2), fullb(BN, 3), fullb(B, N, 3), fullb(B, N, 1),
                  fullb(B, N, 2),
                  fullb(C2, C2), fullb(1, C2), fullb(1, C2), fullb(1, C2),
                  fullb(C2, C3), fullb(1, C3), fullb(1, C3), fullb(1, C3),
                  fullb(C3, 2), fullb(1, 2)],
        out_specs=[fullb(B, 3, 3), fullb(B, 3), fullb(B, N, 2)],
        out_shape=[jax.ShapeDtypeStruct((B, 3, 3), F32),
                   jax.ShapeDtypeStruct((B, 3), F32),
                   jax.ShapeDtypeStruct((B, N, 2), F32)],
    )(af, uv, warped_xyz, lidar_z, gt_project,
      m0_W, m0_b.reshape(1, C2), m0_g.reshape(1, C2), m0_be.reshape(1, C2),
      m1_W, m1_b.reshape(1, C3), m1_g.reshape(1, C3), m1_be.reshape(1, C3),
      m2_W, m2_b.reshape(1, 2))

    return (R, t, weights)


# speedup vs baseline: 19.3770x; 1.3365x over previous
import functools

import jax
import jax.numpy as jnp
from jax import lax
from jax.experimental import pallas as pl
from jax.experimental.pallas import tpu as pltpu
from jax.experimental.pallas import tpu_sc as plsc

F32 = jnp.float32
K = 16
QB = 256
C0, C1, C2 = 128, 64, 64


def _dot(a, b):
    return jnp.dot(a, b, preferred_element_type=F32)


def _knn_body(q_ref, st_ref, sr_ref, out_ref):
    q = q_ref[0]
    st = st_ref[0]
    sr = sr_ref[0]
    M = st.shape[1]
    q2 = jnp.sum(q * q, axis=1, keepdims=True)
    s2 = jnp.sum(st * st, axis=0, keepdims=True)
    e = lax.dot_general(q, sr, (((1,), (1,)), ((), ())),
                        preferred_element_type=F32)
    d = (q2 + s2) - 2.0 * e
    base = pl.program_id(0) * M
    NC = M // 128
    DEPTH = 6
    INF = jnp.float32(jnp.inf)
    BIG = jnp.int32(2**30)
    dc = [d[:, c * 128:(c + 1) * 128] for c in range(NC)]

    V, I = [], []
    for _ in range(DEPTH):
        mv = dc[0]
        for c in range(1, NC):
            mv = jnp.minimum(mv, dc[c])
        hits = [dc[c] <= mv for c in range(NC)]
        ii = jnp.where(hits[0], jnp.int32(0), BIG)
        for c in range(1, NC):
            ii = jnp.minimum(ii, jnp.where(hits[c], jnp.int32(c), BIG))
        for c in range(NC):
            dc[c] = jnp.where(hits[c], INF, dc[c])
        V.append(mv)
        I.append(ii)

    lanes = lax.broadcasted_iota(jnp.int32, (d.shape[0], 128), 1)
    picks = []
    for _ in range(K):
        m = jnp.min(V[0], axis=1, keepdims=True)
        head = V[0] <= m
        gsel = jnp.where(head, I[0] * 128 + lanes, BIG)
        g = jnp.min(gsel, axis=1, keepdims=True)
        picks.append(g + base)
        popm = gsel == g
        for i in range(DEPTH - 1):
            V[i] = jnp.where(popm, V[i + 1], V[i])
            I[i] = jnp.where(popm, I[i + 1], I[i])
        V[DEPTH - 1] = jnp.where(popm, INF, V[DEPTH - 1])
    out_ref[0] = jnp.concatenate(picks, axis=1)


def _knn(q_xyz, st, srows):
    B, N, _ = q_xyz.shape
    M = st.shape[2]
    return pl.pallas_call(
        _knn_body,
        grid=(B, N // QB),
        in_specs=[
            pl.BlockSpec((1, QB, 3), lambda b, i: (b, i, 0)),
            pl.BlockSpec((1, 3, M), lambda b, i: (b, 0, 0)),
            pl.BlockSpec((1, M, 3), lambda b, i: (b, 0, 0)),
        ],
        out_specs=pl.BlockSpec((1, QB, K), lambda b, i: (b, i, 0)),
        out_shape=jax.ShapeDtypeStruct((B, N, K), jnp.int32),
    )(q_xyz, st, srows)


def _sc_gather(table, idx_flat):
    n_idx = idx_flat.shape[0]
    idx2 = idx_flat.reshape(1, n_idx)
    mesh = plsc.VectorSubcoreMesh(core_axis_name="c", subcore_axis_name="s")
    win = 128

    @functools.partial(
        pl.kernel,
        out_type=jax.ShapeDtypeStruct((n_idx, 128), F32),
        mesh=mesh,
    )
    def k(t_hbm, i_hbm, o_hbm):
        def body(i_vmem, o_vmem):
            pltpu.sync_copy(t_hbm.at[i_vmem.at[0]], o_vmem)

        pltpu.emit_pipeline(
            body,
            grid=(n_idx // win,),
            in_specs=[pl.BlockSpec((1, win), lambda i: (0, i))],
            out_specs=[pl.BlockSpec((win, 128), lambda i: (i, 0))],
            core_axis_name=("c", "s"),
            dimension_semantics=(pltpu.PARALLEL,),
        )(i_hbm, o_hbm)

    return k(table, idx2)


def _y0_block(q, g, w0a, w0q, w0g, b0):
    qb = q.shape[0]
    wxyz = q[:, 0:3] * q[:, 3:4]
    y0q = _dot(wxyz, w0a) + _dot(q, w0q) + b0
    g2 = g.reshape(qb * K, g.shape[2])
    y0g = _dot(g2, w0g)
    return y0g + jnp.broadcast_to(y0q[:, None, :], (qb, K, C0)).reshape(qb * K, C0)


def _acc_stats(i, s_ref, ss_ref, y):
    @pl.when(i == 0)
    def _():
        s_ref[...] = jnp.zeros_like(s_ref)
        ss_ref[...] = jnp.zeros_like(ss_ref)

    s_ref[...] += jnp.sum(y, axis=0, keepdims=True)
    ss_ref[...] += jnp.sum(y * y, axis=0, keepdims=True)


def _bn_relu(y, s, ss, gamma, beta, count):
    mean = s / count
    var = jnp.maximum(ss / count - mean * mean, 0.0)
    return jnp.maximum((y - mean) * lax.rsqrt(var + 1e-5) * gamma + beta, 0.0)


def _stats0_body(g_ref, q_ref, w0a_ref, w0q_ref, w0g_ref, b0_ref,
                 s_ref, ss_ref):
    y0 = _y0_block(q_ref[...], g_ref[...], w0a_ref[...], w0q_ref[...],
                   w0g_ref[...], b0_ref[...])
    _acc_stats(pl.program_id(0), s_ref, ss_ref, y0)


def _l1_body(g_ref, q_ref, w0a_ref, w0q_ref, w0g_ref, b0_ref,
             s0_ref, ss0_ref, g0_ref, be0_ref, w1_ref, b1_ref,
             y1_ref, gx_ref, s_ref, ss_ref, *, count):
    gx_ref[...] = g_ref[...][:, :, 64:72]
    y0 = _y0_block(q_ref[...], g_ref[...], w0a_ref[...], w0q_ref[...],
                   w0g_ref[...], b0_ref[...])
    x1 = _bn_relu(y0, s0_ref[...], ss0_ref[...], g0_ref[...], be0_ref[...],
                  count)
    y1 = _dot(x1, w1_ref[...]) + b1_ref[...]
    y1_ref[...] = y1.reshape(QB, K, C1)
    _acc_stats(pl.program_id(0), s_ref, ss_ref, y1)


def _l2_body(y1_ref, s1_ref, ss1_ref, g1_ref, be1_ref, w2_ref, b2_ref,
             y2_ref, s_ref, ss_ref, *, count):
    y1 = y1_ref[...].reshape(QB * K, C1)
    x2 = _bn_relu(y1, s1_ref[...], ss1_ref[...], g1_ref[...], be1_ref[...],
                  count)
    y2 = _dot(x2, w2_ref[...]) + b2_ref[...]
    y2_ref[...] = y2.reshape(QB, K, C2)
    _acc_stats(pl.program_id(0), s_ref, ss_ref, y2)


def _agg_body(y2_ref, gx_ref, s2_ref, ss2_ref, g2_ref, be2_ref,
              af_ref, uv_ref, *, count):
    y2 = y2_ref[...].reshape(QB * K, C2)
    feats = _bn_relu(y2, s2_ref[...], ss2_ref[...], g2_ref[...], be2_ref[...],
                     count).reshape(QB, K, C2)
    mx = jnp.max(feats, axis=2)
    e = jnp.exp(mx - jnp.max(mx, axis=1, keepdims=True))
    aw = e / jnp.sum(e, axis=1, keepdims=True)
    af_ref[...] = jnp.sum(aw[:, :, None] * feats, axis=1)
    gx = gx_ref[...][:, :, 0:3]
    uv_ref[...] = jnp.sum(aw[:, :, None] * gx, axis=1)


def _mask(p, q):
    r = lax.broadcasted_iota(jnp.int32, (3, 3), 0)
    c = lax.broadcasted_iota(jnp.int32, (3, 3), 1)
    return ((r == p) & (c == q)).astype(F32)


def _jacobi_svd_r_t(Hm, cs, ct):
    eye = _mask(0, 0) + _mask(1, 1) + _mask(2, 2)
    A = lax.dot_general(Hm, Hm, (((0,), (0,)), ((), ())))
    V = eye
    for _ in range(6):
        for (p, q) in ((0, 1), (0, 2), (1, 2)):
            app = A[p:p + 1, p:p + 1]
            aqq = A[q:q + 1, q:q + 1]
            apq = A[p:p + 1, q:q + 1]
            nz = jnp.abs(apq) > 1e-30
            apq_s = jnp.where(nz, apq, 1.0)
            tau = (aqq - app) / (2.0 * apq_s)
            sg = jnp.where(tau >= 0.0, 1.0, -1.0)
            t = sg / (jnp.abs(tau) + jnp.sqrt(1.0 + tau * tau))
            t = jnp.where(nz, t, 0.0)
            c = lax.rsqrt(1.0 + t * t)
            s = t * c
            J = eye + (c - 1.0) * (_mask(p, p) + _mask(q, q)) \
                + s * _mask(p, q) - s * _mask(q, p)
            A = _dot(lax.dot_general(J, A, (((0,), (0,)), ((), ()))), J)
            V = _dot(V, J)
    l0 = A[0:1, 0:1]
    l1 = A[1:2, 1:2]
    l2 = A[2:3, 2:3]
    detv = jnp.ones_like(l0)
    def swap(li, lj, V, detv, i, j):
        cnd = li < lj
        P = eye - _mask(i, i) - _mask(j, j) + _mask(i, j) + _mask(j, i)
        Vn = jnp.where(cnd, _dot(V, P), V)
        dn = jnp.where(cnd, -detv, detv)
        return jnp.where(cnd, lj, li), jnp.where(cnd, li, lj), Vn, dn
    l0, l1, V, detv = swap(l0, l1, V, detv, 0, 1)
    l0, l2, V, detv = swap(l0, l2, V, detv, 0, 2)
    l1, l2, V, detv = swap(l1, l2, V, detv, 1, 2)
    HV = _dot(Hm, V)
    norm2 = jnp.sum(HV * HV, axis=0, keepdims=True)
    Un = HV * lax.rsqrt(norm2 + 1e-30)
    a = Un[:, 0:1]
    b = Un[:, 1:2]
    u2 = jnp.concatenate(
        [a[1:2] * b[2:3] - a[2:3] * b[1:2],
         a[2:3] * b[0:1] - a[0:1] * b[2:3],
         a[0:1] * b[1:2] - a[1:2] * b[0:1]], axis=0)
    U = jnp.concatenate([a, b, u2], axis=1)
    d_row = jnp.concatenate([jnp.ones_like(detv), jnp.ones_like(detv), detv],
                            axis=1)
    R = lax.dot_general(V * d_row, U, (((1,), (1,)), ((), ())))
    t = ct - lax.dot_general(cs, R, (((1,), (1,)), ((), ())))
    return R, t


def _head_body(af_ref, uv_ref, xyz_ref, z_ref, gt_ref,
               m0w_ref, m0b_ref, m0g_ref, m0be_ref,
               m1w_ref, m1b_ref, m1g_ref, m1be_ref,
               m2w_ref, m2b_ref,
               r_ref, t_ref, w_ref, *, n_rows, n_pts):
    a = af_ref[...]
    y = _dot(a, m0w_ref[...]) + m0b_ref[...]
    s = jnp.sum(y, axis=0, keepdims=True)
    ss = jnp.sum(y * y, axis=0, keepdims=True)
    x = _bn_relu(y, s, ss, m0g_ref[...], m0be_ref[...], float(n_rows))
    y = _dot(x, m1w_ref[...]) + m1b_ref[...]
    s = jnp.sum(y, axis=0, keepdims=True)
    ss = jnp.sum(y * y, axis=0, keepdims=True)
    x = _bn_relu(y, s, ss, m1g_ref[...], m1be_ref[...], float(n_rows))
    w = _dot(x, m2w_ref[...]) + m2b_ref[...]
    w_ref[...] = w.reshape(w_ref.shape)

    for b in range(w_ref.shape[0]):
        gt = gt_ref[b]
        win = jnp.where(gt[:, 1:2] > gt[:, 0:1], 1.0, 0.0)
        wn = win / (jnp.sum(win) + 1e-8)
        src = xyz_ref[b] * z_ref[b]
        uv = uv_ref[pl.ds(b * n_pts, n_pts), 0:2]
        tgt = jnp.concatenate([uv, jnp.ones_like(uv[:, 0:1])], axis=1)
        cs = jnp.sum(wn * src, axis=0, keepdims=True)
        ct = jnp.sum(wn * tgt, axis=0, keepdims=True)
        sc = src - cs
        tc = tgt - ct
        Hm = lax.dot_general(wn * sc, tc, (((0,), (0,)), ((), ())))
        R, t = _jacobi_svd_r_t(Hm, cs, ct)
        r_ref[b] = R
        t_ref[pl.ds(b, 1), :] = t


def kernel(warped_xyz, warped_points, RF3, RF3_index, lidar_z, gt_project,
           c0_W, c0_b, c0_g, c0_be, c1_W, c1_b, c1_g, c1_be,
           c2_W, c2_b, c2_g, c2_be, m0_W, m0_b, m0_g, m0_be,
           m1_W, m1_b, m1_g, m1_be, m2_W, m2_b):
    B, N, _ = warped_xyz.shape
    _, Ci, H, W_ = RF3.shape
    M = H * W_
    BN = B * N
    rows = BN * K
    count = float(rows)

    st = RF3_index.reshape(B, 3, M)
    srows = RF3_index.transpose(0, 2, 3, 1).reshape(B, M, 3)
    idx = _knn(warped_xyz, st, srows)

    feat_t = RF3.transpose(0, 2, 3, 1).reshape(B * M, Ci)
    xyz_t = srows.reshape(B * M, 3)
    table = jnp.concatenate(
        [feat_t, xyz_t, jnp.zeros((B * M, 61), F32)], axis=1)

    g_rows = _sc_gather(table, idx.reshape(rows))
    g3 = g_rows.reshape(BN, K, 128)

    q_cat = jnp.concatenate(
        [warped_xyz.reshape(BN, 3), lidar_z.reshape(BN, 1),
         warped_points.reshape(BN, Ci)], axis=1)

    w0a = c0_W[0:3]
    w0q = jnp.concatenate([jnp.zeros((4, C0), F32), c0_W[6:70]], axis=0)
    w0g = jnp.concatenate([c0_W[70:134], c0_W[3:6],
                           jnp.zeros((61, C0), F32)], axis=0)

    nblk = BN // QB
    g_spec = pl.BlockSpec((QB, K, 128), lambda i: (i, 0, 0))
    q_spec = pl.BlockSpec((QB, 68), lambda i: (i, 0))
    full = lambda *shape: pl.BlockSpec(shape, lambda i: tuple(0 for _ in shape))
    acc_spec = lambda c: pl.BlockSpec((1, c), lambda i: (0, 0))

    s0, ss0 = pl.pallas_call(
        _stats0_body,
        grid=(nblk,),
        in_specs=[g_spec, q_spec, full(3, C0), full(68, C0), full(128, C0),
                  full(1, C0)],
        out_specs=[acc_spec(C0), acc_spec(C0)],
        out_shape=[jax.ShapeDtypeStruct((1, C0), F32)] * 2,
    )(g3, q_cat, w0a, w0q, w0g, c0_b.reshape(1, C0))

    y1, gx3, s1, ss1 = pl.pallas_call(
        functools.partial(_l1_body, count=count),
        grid=(nblk,),
        in_specs=[g_spec, q_spec, full(3, C0), full(68, C0), full(128, C0),
                  full(1, C0), full(1, C0), full(1, C0), full(1, C0),
                  full(1, C0), full(C0, C1), full(1, C1)],
        out_specs=[pl.BlockSpec((QB, K, C1), lambda i: (i, 0, 0)),
                   pl.BlockSpec((QB, K, 8), lambda i: (i, 0, 0)),
                   acc_spec(C1), acc_spec(C1)],
        out_shape=[jax.ShapeDtypeStruct((BN, K, C1), F32),
                   jax.ShapeDtypeStruct((BN, K, 8), F32),
                   jax.ShapeDtypeStruct((1, C1), F32),
                   jax.ShapeDtypeStruct((1, C1), F32)],
    )(g3, q_cat, w0a, w0q, w0g, c0_b.reshape(1, C0), s0, ss0,
      c0_g.reshape(1, C0), c0_be.reshape(1, C0), c1_W, c1_b.reshape(1, C1))

    y2, s2, ss2 = pl.pallas_call(
        functools.partial(_l2_body, count=count),
        grid=(nblk,),
        in_specs=[pl.BlockSpec((QB, K, C1), lambda i: (i, 0, 0)),
                  full(1, C1), full(1, C1), full(1, C1), full(1, C1),
                  full(C1, C2), full(1, C2)],
        out_specs=[pl.BlockSpec((QB, K, C2), lambda i: (i, 0, 0)),
                   acc_spec(C2), acc_spec(C2)],
        out_shape=[jax.ShapeDtypeStruct((BN, K, C2), F32),
                   jax.ShapeDtypeStruct((1, C2), F32),
                   jax.ShapeDtypeStruct((1, C2), F32)],
    )(y1, s1, ss1, c1_g.reshape(1, C1), c1_be.reshape(1, C1),
      c2_W, c2_b.reshape(1, C2))

    af, uv = pl.pallas_call(
        functools.partial(_agg_body, count=count),
        grid=(nblk,),
        in_specs=[pl.BlockSpec((QB, K, C2), lambda i: (i, 0, 0)),
                  pl.BlockSpec((QB, K, 8), lambda i: (i, 0, 0)),
                  full(1, C2), full(1, C2), full(1, C2), full(1, C2)],
        out_specs=[pl.BlockSpec((QB, C2), lambda i: (i, 0)),
                   pl.BlockSpec((QB, 3), lambda i: (i, 0))],
        out_shape=[jax.ShapeDtypeStruct((BN, C2), F32),
                   jax.ShapeDtypeStruct((BN, 3), F32)],
    )(y2, gx3, s2, ss2, c2_g.reshape(1, C2), c2_be.reshape(1, C2))

    C3 = m1_W.shape[1]
    fullb = lambda *shape: pl.BlockSpec(shape, lambda: tuple(0 for _ in shape))
    R, t, weights = pl.pallas_call(
        functools.partial(_head_body, n_rows=BN, n_pts=N),
        in_specs=[fullb(BN, C2), fullb(BN, 3), fullb(B, N, 3), fullb(B, N, 1),
                  fullb(B, N, 2),
                  fullb(C2, C2), fullb(1, C2), fullb(1, C2), fullb(1, C2),
                  fullb(C2, C3), fullb(1, C3), fullb(1, C3), fullb(1, C3),
                  fullb(C3, 2), fullb(1, 2)],
        out_specs=[fullb(B, 3, 3), fullb(B, 3), fullb(B, N, 2)],
        out_shape=[jax.ShapeDtypeStruct((B, 3, 3), F32),
                   jax.ShapeDtypeStruct((B, 3), F32),
                   jax.ShapeDtypeStruct((B, N, 2), F32)],
    )(af, uv, warped_xyz, lidar_z, gt_project,
      m0_W, m0_b.reshape(1, C2), m0_g.reshape(1, C2), m0_be.reshape(1, C2),
      m1_W, m1_b.reshape(1, C3), m1_g.reshape(1, C3), m1_be.reshape(1, C3),
      m2_W, m2_b.reshape(1, 2))

    return (R, t, weights)


# speedup vs baseline: 21.2563x; 1.0970x over previous
import functools

import jax
import jax.numpy as jnp
from jax import lax
from jax.experimental import pallas as pl
from jax.experimental.pallas import tpu as pltpu
from jax.experimental.pallas import tpu_sc as plsc

F32 = jnp.float32
K = 16
QB = 256
C0, C1, C2 = 128, 64, 64


def _dot(a, b):
    return jnp.dot(a, b, preferred_element_type=F32)


def _knn_body(q_ref, st_ref, sr_ref, out_ref):
    q = q_ref[0]
    st = st_ref[0]
    sr = sr_ref[0]
    M = st.shape[1]
    q2 = jnp.sum(q * q, axis=1, keepdims=True)
    s2 = jnp.sum(st * st, axis=0, keepdims=True)
    e = lax.dot_general(q, sr, (((1,), (1,)), ((), ())),
                        preferred_element_type=F32)
    d = (q2 + s2) - 2.0 * e
    base = pl.program_id(0) * M
    NC = M // 128
    DEPTH = 6
    INF = jnp.float32(jnp.inf)
    BIG = jnp.int32(2**30)
    V = [jnp.full(d.shape[:1] + (128,), INF, F32) for _ in range(DEPTH)]
    I = [jnp.zeros(d.shape[:1] + (128,), jnp.int32) for _ in range(DEPTH)]
    for c in range(NC):
        x = d[:, c * 128:(c + 1) * 128]
        xi = jnp.full(x.shape, jnp.int32(c), jnp.int32)
        for t in range(DEPTH):
            lt = x < V[t]
            nv = jnp.minimum(V[t], x)
            x = jnp.maximum(V[t], x)
            ni = jnp.where(lt, xi, I[t])
            xi = jnp.where(lt, I[t], xi)
            V[t] = nv
            I[t] = ni

    lanes = lax.broadcasted_iota(jnp.int32, (d.shape[0], 128), 1)
    picks = []
    for _ in range(K):
        m = jnp.min(V[0], axis=1, keepdims=True)
        head = V[0] <= m
        gsel = jnp.where(head, I[0] * 128 + lanes, BIG)
        g = jnp.min(gsel, axis=1, keepdims=True)
        picks.append(g + base)
        popm = gsel == g
        for i in range(DEPTH - 1):
            V[i] = jnp.where(popm, V[i + 1], V[i])
            I[i] = jnp.where(popm, I[i + 1], I[i])
        V[DEPTH - 1] = jnp.where(popm, INF, V[DEPTH - 1])
    out_ref[0] = jnp.concatenate(picks, axis=1)


def _knn(q_xyz, st, srows):
    B, N, _ = q_xyz.shape
    M = st.shape[2]
    return pl.pallas_call(
        _knn_body,
        grid=(B, N // QB),
        in_specs=[
            pl.BlockSpec((1, QB, 3), lambda b, i: (b, i, 0)),
            pl.BlockSpec((1, 3, M), lambda b, i: (b, 0, 0)),
            pl.BlockSpec((1, M, 3), lambda b, i: (b, 0, 0)),
        ],
        out_specs=pl.BlockSpec((1, QB, K), lambda b, i: (b, i, 0)),
        out_shape=jax.ShapeDtypeStruct((B, N, K), jnp.int32),
    )(q_xyz, st, srows)


def _sc_gather(table, idx_flat):
    n_idx = idx_flat.shape[0]
    idx2 = idx_flat.reshape(1, n_idx)
    mesh = plsc.VectorSubcoreMesh(core_axis_name="c", subcore_axis_name="s")
    win = 128

    @functools.partial(
        pl.kernel,
        out_type=jax.ShapeDtypeStruct((n_idx, 128), F32),
        mesh=mesh,
    )
    def k(t_hbm, i_hbm, o_hbm):
        def body(i_vmem, o_vmem):
            pltpu.sync_copy(t_hbm.at[i_vmem.at[0]], o_vmem)

        pltpu.emit_pipeline(
            body,
            grid=(n_idx // win,),
            in_specs=[pl.BlockSpec((1, win), lambda i: (0, i))],
            out_specs=[pl.BlockSpec((win, 128), lambda i: (i, 0))],
            core_axis_name=("c", "s"),
            dimension_semantics=(pltpu.PARALLEL,),
        )(i_hbm, o_hbm)

    return k(table, idx2)


def _y0_block(q, g, w0a, w0q, w0g, b0):
    qb = q.shape[0]
    wxyz = q[:, 0:3] * q[:, 3:4]
    y0q = _dot(wxyz, w0a) + _dot(q, w0q) + b0
    g2 = g.reshape(qb * K, g.shape[2])
    y0g = _dot(g2, w0g)
    return y0g + jnp.broadcast_to(y0q[:, None, :], (qb, K, C0)).reshape(qb * K, C0)


def _acc_stats(i, s_ref, ss_ref, y):
    @pl.when(i == 0)
    def _():
        s_ref[...] = jnp.zeros_like(s_ref)
        ss_ref[...] = jnp.zeros_like(ss_ref)

    s_ref[...] += jnp.sum(y, axis=0, keepdims=True)
    ss_ref[...] += jnp.sum(y * y, axis=0, keepdims=True)


def _bn_relu(y, s, ss, gamma, beta, count):
    mean = s / count
    var = jnp.maximum(ss / count - mean * mean, 0.0)
    return jnp.maximum((y - mean) * lax.rsqrt(var + 1e-5) * gamma + beta, 0.0)


def _stats0_body(g_ref, q_ref, w0a_ref, w0q_ref, w0g_ref, b0_ref,
                 s_ref, ss_ref):
    y0 = _y0_block(q_ref[...], g_ref[...], w0a_ref[...], w0q_ref[...],
                   w0g_ref[...], b0_ref[...])
    _acc_stats(pl.program_id(0), s_ref, ss_ref, y0)


def _l1_body(g_ref, q_ref, w0a_ref, w0q_ref, w0g_ref, b0_ref,
             s0_ref, ss0_ref, g0_ref, be0_ref, w1_ref, b1_ref,
             y1_ref, gx_ref, s_ref, ss_ref, *, count):
    gx_ref[...] = g_ref[...][:, :, 64:72]
    y0 = _y0_block(q_ref[...], g_ref[...], w0a_ref[...], w0q_ref[...],
                   w0g_ref[...], b0_ref[...])
    x1 = _bn_relu(y0, s0_ref[...], ss0_ref[...], g0_ref[...], be0_ref[...],
                  count)
    y1 = _dot(x1, w1_ref[...]) + b1_ref[...]
    y1_ref[...] = y1.reshape(QB, K, C1)
    _acc_stats(pl.program_id(0), s_ref, ss_ref, y1)


def _l2_body(y1_ref, s1_ref, ss1_ref, g1_ref, be1_ref, w2_ref, b2_ref,
             y2_ref, s_ref, ss_ref, *, count):
    y1 = y1_ref[...].reshape(QB * K, C1)
    x2 = _bn_relu(y1, s1_ref[...], ss1_ref[...], g1_ref[...], be1_ref[...],
                  count)
    y2 = _dot(x2, w2_ref[...]) + b2_ref[...]
    y2_ref[...] = y2.reshape(QB, K, C2)
    _acc_stats(pl.program_id(0), s_ref, ss_ref, y2)


def _agg_body(y2_ref, gx_ref, s2_ref, ss2_ref, g2_ref, be2_ref,
              af_ref, uv_ref, *, count):
    y2 = y2_ref[...].reshape(QB * K, C2)
    feats = _bn_relu(y2, s2_ref[...], ss2_ref[...], g2_ref[...], be2_ref[...],
                     count).reshape(QB, K, C2)
    mx = jnp.max(feats, axis=2)
    e = jnp.exp(mx - jnp.max(mx, axis=1, keepdims=True))
    aw = e / jnp.sum(e, axis=1, keepdims=True)
    af_ref[...] = jnp.sum(aw[:, :, None] * feats, axis=1)
    gx = gx_ref[...][:, :, 0:3]
    uv_ref[...] = jnp.sum(aw[:, :, None] * gx, axis=1)


def _mask(p, q):
    r = lax.broadcasted_iota(jnp.int32, (3, 3), 0)
    c = lax.broadcasted_iota(jnp.int32, (3, 3), 1)
    return ((r == p) & (c == q)).astype(F32)


def _jacobi_svd_r_t(Hm, cs, ct):
    eye = _mask(0, 0) + _mask(1, 1) + _mask(2, 2)
    A = lax.dot_general(Hm, Hm, (((0,), (0,)), ((), ())))
    V = eye
    for _ in range(6):
        for (p, q) in ((0, 1), (0, 2), (1, 2)):
            app = A[p:p + 1, p:p + 1]
            aqq = A[q:q + 1, q:q + 1]
            apq = A[p:p + 1, q:q + 1]
            nz = jnp.abs(apq) > 1e-30
            apq_s = jnp.where(nz, apq, 1.0)
            tau = (aqq - app) / (2.0 * apq_s)
            sg = jnp.where(tau >= 0.0, 1.0, -1.0)
            t = sg / (jnp.abs(tau) + jnp.sqrt(1.0 + tau * tau))
            t = jnp.where(nz, t, 0.0)
            c = lax.rsqrt(1.0 + t * t)
            s = t * c
            J = eye + (c - 1.0) * (_mask(p, p) + _mask(q, q)) \
                + s * _mask(p, q) - s * _mask(q, p)
            A = _dot(lax.dot_general(J, A, (((0,), (0,)), ((), ()))), J)
            V = _dot(V, J)
    l0 = A[0:1, 0:1]
    l1 = A[1:2, 1:2]
    l2 = A[2:3, 2:3]
    detv = jnp.ones_like(l0)
    def swap(li, lj, V, detv, i, j):
        cnd = li < lj
        P = eye - _mask(i, i) - _mask(j, j) + _mask(i, j) + _mask(j, i)
        Vn = jnp.where(cnd, _dot(V, P), V)
        dn = jnp.where(cnd, -detv, detv)
        return jnp.where(cnd, lj, li), jnp.where(cnd, li, lj), Vn, dn
    l0, l1, V, detv = swap(l0, l1, V, detv, 0, 1)
    l0, l2, V, detv = swap(l0, l2, V, detv, 0, 2)
    l1, l2, V, detv = swap(l1, l2, V, detv, 1, 2)
    HV = _dot(Hm, V)
    norm2 = jnp.sum(HV * HV, axis=0, keepdims=True)
    Un = HV * lax.rsqrt(norm2 + 1e-30)
    a = Un[:, 0:1]
    b = Un[:, 1:2]
    u2 = jnp.concatenate(
        [a[1:2] * b[2:3] - a[2:3] * b[1:2],
         a[2:3] * b[0:1] - a[0:1] * b[2:3],
         a[0:1] * b[1:2] - a[1:2] * b[0:1]], axis=0)
    U = jnp.concatenate([a, b, u2], axis=1)
    d_row = jnp.concatenate([jnp.ones_like(detv), jnp.ones_like(detv), detv],
                            axis=1)
    R = lax.dot_general(V * d_row, U, (((1,), (1,)), ((), ())))
    t = ct - lax.dot_general(cs, R, (((1,), (1,)), ((), ())))
    return R, t


def _head_body(af_ref, uv_ref, xyz_ref, z_ref, gt_ref,
               m0w_ref, m0b_ref, m0g_ref, m0be_ref,
               m1w_ref, m1b_ref, m1g_ref, m1be_ref,
               m2w_ref, m2b_ref,
               r_ref, t_ref, w_ref, *, n_rows, n_pts):
    a = af_ref[...]
    y = _dot(a, m0w_ref[...]) + m0b_ref[...]
    s = jnp.sum(y, axis=0, keepdims=True)
    ss = jnp.sum(y * y, axis=0, keepdims=True)
    x = _bn_relu(y, s, ss, m0g_ref[...], m0be_ref[...], float(n_rows))
    y = _dot(x, m1w_ref[...]) + m1b_ref[...]
    s = jnp.sum(y, axis=0, keepdims=True)
    ss = jnp.sum(y * y, axis=0, keepdims=True)
    x = _bn_relu(y, s, ss, m1g_ref[...], m1be_ref[...], float(n_rows))
    w = _dot(x, m2w_ref[...]) + m2b_ref[...]
    w_ref[...] = w.reshape(w_ref.shape)

    for b in range(w_ref.shape[0]):
        gt = gt_ref[b]
        win = jnp.where(gt[:, 1:2] > gt[:, 0:1], 1.0, 0.0)
        wn = win / (jnp.sum(win) + 1e-8)
        src = xyz_ref[b] * z_ref[b]
        uv = uv_ref[pl.ds(b * n_pts, n_pts), 0:2]
        tgt = jnp.concatenate([uv, jnp.ones_like(uv[:, 0:1])], axis=1)
        cs = jnp.sum(wn * src, axis=0, keepdims=True)
        ct = jnp.sum(wn * tgt, axis=0, keepdims=True)
        sc = src - cs
        tc = tgt - ct
        Hm = lax.dot_general(wn * sc, tc, (((0,), (0,)), ((), ())))
        R, t = _jacobi_svd_r_t(Hm, cs, ct)
        r_ref[b] = R
        t_ref[pl.ds(b, 1), :] = t


def kernel(warped_xyz, warped_points, RF3, RF3_index, lidar_z, gt_project,
           c0_W, c0_b, c0_g, c0_be, c1_W, c1_b, c1_g, c1_be,
           c2_W, c2_b, c2_g, c2_be, m0_W, m0_b, m0_g, m0_be,
           m1_W, m1_b, m1_g, m1_be, m2_W, m2_b):
    B, N, _ = warped_xyz.shape
    _, Ci, H, W_ = RF3.shape
    M = H * W_
    BN = B * N
    rows = BN * K
    count = float(rows)

    st = RF3_index.reshape(B, 3, M)
    srows = RF3_index.transpose(0, 2, 3, 1).reshape(B, M, 3)

    feat_t = RF3.transpose(0, 2, 3, 1).reshape(B * M, Ci)
    xyz_t = srows.reshape(B * M, 3)
    table = jnp.concatenate(
        [feat_t, xyz_t, jnp.zeros((B * M, 61), F32)], axis=1)

    idx = _knn(warped_xyz, st, srows)
    g_rows = _sc_gather(table, idx.reshape(rows))
    g3 = g_rows.reshape(BN, K, 128)

    q_cat = jnp.concatenate(
        [warped_xyz.reshape(BN, 3), lidar_z.reshape(BN, 1),
         warped_points.reshape(BN, Ci)], axis=1)

    w0a = c0_W[0:3]
    w0q = jnp.concatenate([jnp.zeros((4, C0), F32), c0_W[6:70]], axis=0)
    w0g = jnp.concatenate([c0_W[70:134], c0_W[3:6],
                           jnp.zeros((61, C0), F32)], axis=0)

    nblk = BN // QB
    g_spec = pl.BlockSpec((QB, K, 128), lambda i: (i, 0, 0))
    q_spec = pl.BlockSpec((QB, 68), lambda i: (i, 0))
    full = lambda *shape: pl.BlockSpec(shape, lambda i: tuple(0 for _ in shape))
    acc_spec = lambda c: pl.BlockSpec((1, c), lambda i: (0, 0))

    s0, ss0 = pl.pallas_call(
        _stats0_body,
        grid=(nblk,),
        in_specs=[g_spec, q_spec, full(3, C0), full(68, C0), full(128, C0),
                  full(1, C0)],
        out_specs=[acc_spec(C0), acc_spec(C0)],
        out_shape=[jax.ShapeDtypeStruct((1, C0), F32)] * 2,
    )(g3, q_cat, w0a, w0q, w0g, c0_b.reshape(1, C0))

    y1, gx3, s1, ss1 = pl.pallas_call(
        functools.partial(_l1_body, count=count),
        grid=(nblk,),
        in_specs=[g_spec, q_spec, full(3, C0), full(68, C0), full(128, C0),
                  full(1, C0), full(1, C0), full(1, C0), full(1, C0),
                  full(1, C0), full(C0, C1), full(1, C1)],
        out_specs=[pl.BlockSpec((QB, K, C1), lambda i: (i, 0, 0)),
                   pl.BlockSpec((QB, K, 8), lambda i: (i, 0, 0)),
                   acc_spec(C1), acc_spec(C1)],
        out_shape=[jax.ShapeDtypeStruct((BN, K, C1), F32),
                   jax.ShapeDtypeStruct((BN, K, 8), F32),
                   jax.ShapeDtypeStruct((1, C1), F32),
                   jax.ShapeDtypeStruct((1, C1), F32)],
    )(g3, q_cat, w0a, w0q, w0g, c0_b.reshape(1, C0), s0, ss0,
      c0_g.reshape(1, C0), c0_be.reshape(1, C0), c1_W, c1_b.reshape(1, C1))

    y2, s2, ss2 = pl.pallas_call(
        functools.partial(_l2_body, count=count),
        grid=(nblk,),
        in_specs=[pl.BlockSpec((QB, K, C1), lambda i: (i, 0, 0)),
                  full(1, C1), full(1, C1), full(1, C1), full(1, C1),
                  full(C1, C2), full(1, C2)],
        out_specs=[pl.BlockSpec((QB, K, C2), lambda i: (i, 0, 0)),
                   acc_spec(C2), acc_spec(C2)],
        out_shape=[jax.ShapeDtypeStruct((BN, K, C2), F32),
                   jax.ShapeDtypeStruct((1, C2), F32),
                   jax.ShapeDtypeStruct((1, C2), F32)],
    )(y1, s1, ss1, c1_g.reshape(1, C1), c1_be.reshape(1, C1),
      c2_W, c2_b.reshape(1, C2))

    af, uv = pl.pallas_call(
        functools.partial(_agg_body, count=count),
        grid=(nblk,),
        in_specs=[pl.BlockSpec((QB, K, C2), lambda i: (i, 0, 0)),
                  pl.BlockSpec((QB, K, 8), lambda i: (i, 0, 0)),
                  full(1, C2), full(1, C2), full(1, C2), full(1, C2)],
        out_specs=[pl.BlockSpec((QB, C2), lambda i: (i, 0)),
                   pl.BlockSpec((QB, 3), lambda i: (i, 0))],
        out_shape=[jax.ShapeDtypeStruct((BN, C2), F32),
                   jax.ShapeDtypeStruct((BN, 3), F32)],
    )(y2, gx3, s2, ss2, c2_g.reshape(1, C2), c2_be.reshape(1, C2))

    C3 = m1_W.shape[1]
    fullb = lambda *shape: pl.BlockSpec(shape, lambda: tuple(0 for _ in shape))
    R, t, weights = pl.pallas_call(
        functools.partial(_head_body, n_rows=BN, n_pts=N),
        in_specs=[fullb(BN, C2), fullb(BN, 3), fullb(B, N, 3), fullb(B, N, 1),
                  fullb(B, N, 2),
                  fullb(C2, C2), fullb(1, C2), fullb(1, C2), fullb(1, C2),
                  fullb(C2, C3), fullb(1, C3), fullb(1, C3), fullb(1, C3),
                  fullb(C3, 2), fullb(1, 2)],
        out_specs=[fullb(B, 3, 3), fullb(B, 3), fullb(B, N, 2)],
        out_shape=[jax.ShapeDtypeStruct((B, 3, 3), F32),
                   jax.ShapeDtypeStruct((B, 3), F32),
                   jax.ShapeDtypeStruct((B, N, 2), F32)],
    )(af, uv, warped_xyz, lidar_z, gt_project,
      m0_W, m0_b.reshape(1, C2), m0_g.reshape(1, C2), m0_be.reshape(1, C2),
      m1_W, m1_b.reshape(1, C3), m1_g.reshape(1, C3), m1_be.reshape(1, C3),
      m2_W, m2_b.reshape(1, 2))

    return (R, t, weights)


# speedup vs baseline: 23.0343x; 1.0836x over previous
import functools

import jax
import jax.numpy as jnp
from jax import lax
from jax.experimental import pallas as pl
from jax.experimental.pallas import tpu as pltpu
from jax.experimental.pallas import tpu_sc as plsc

F32 = jnp.float32
K = 16
QB = 256
C0, C1, C2 = 128, 64, 64


def _dot(a, b):
    return jnp.dot(a, b, preferred_element_type=F32)


def _knn_body(q_ref, st_ref, sr_ref, out_ref):
    q = q_ref[0]
    st = st_ref[0]
    sr = sr_ref[0]
    M = st.shape[1]
    q2 = jnp.sum(q * q, axis=1, keepdims=True)
    s2 = jnp.sum(st * st, axis=0, keepdims=True)
    e = lax.dot_general(q, sr, (((1,), (1,)), ((), ())),
                        preferred_element_type=F32)
    d = (q2 + s2) - 2.0 * e
    base = pl.program_id(0) * M
    NC = M // 128
    DEPTH = 4
    INF = jnp.float32(jnp.inf)
    BIG = jnp.int32(2**30)
    V = [jnp.full(d.shape[:1] + (128,), INF, F32) for _ in range(DEPTH)]
    I = [jnp.zeros(d.shape[:1] + (128,), jnp.int32) for _ in range(DEPTH)]
    for c in range(NC):
        x = d[:, c * 128:(c + 1) * 128]
        xi = jnp.full(x.shape, jnp.int32(c), jnp.int32)
        for t in range(DEPTH):
            lt = x < V[t]
            nv = jnp.minimum(V[t], x)
            x = jnp.maximum(V[t], x)
            ni = jnp.where(lt, xi, I[t])
            xi = jnp.where(lt, I[t], xi)
            V[t] = nv
            I[t] = ni

    lanes = lax.broadcasted_iota(jnp.int32, (d.shape[0], 128), 1)
    picks = []
    for _ in range(K):
        m = jnp.min(V[0], axis=1, keepdims=True)
        head = V[0] <= m
        gsel = jnp.where(head, I[0] * 128 + lanes, BIG)
        g = jnp.min(gsel, axis=1, keepdims=True)
        picks.append(g + base)
        popm = gsel == g
        for i in range(DEPTH - 1):
            V[i] = jnp.where(popm, V[i + 1], V[i])
            I[i] = jnp.where(popm, I[i + 1], I[i])
        V[DEPTH - 1] = jnp.where(popm, INF, V[DEPTH - 1])
    out_ref[0] = jnp.concatenate(picks, axis=1)


def _knn(q_xyz, st, srows):
    B, N, _ = q_xyz.shape
    M = st.shape[2]
    return pl.pallas_call(
        _knn_body,
        grid=(B, N // QB),
        in_specs=[
            pl.BlockSpec((1, QB, 3), lambda b, i: (b, i, 0)),
            pl.BlockSpec((1, 3, M), lambda b, i: (b, 0, 0)),
            pl.BlockSpec((1, M, 3), lambda b, i: (b, 0, 0)),
        ],
        out_specs=pl.BlockSpec((1, QB, K), lambda b, i: (b, i, 0)),
        out_shape=jax.ShapeDtypeStruct((B, N, K), jnp.int32),
    )(q_xyz, st, srows)


def _sc_gather(table, idx_flat):
    n_idx = idx_flat.shape[0]
    idx2 = idx_flat.reshape(1, n_idx)
    mesh = plsc.VectorSubcoreMesh(core_axis_name="c", subcore_axis_name="s")
    win = 128

    @functools.partial(
        pl.kernel,
        out_type=jax.ShapeDtypeStruct((n_idx, 128), F32),
        mesh=mesh,
    )
    def k(t_hbm, i_hbm, o_hbm):
        def body(i_vmem, o_vmem):
            pltpu.sync_copy(t_hbm.at[i_vmem.at[0]], o_vmem)

        pltpu.emit_pipeline(
            body,
            grid=(n_idx // win,),
            in_specs=[pl.BlockSpec((1, win), lambda i: (0, i))],
            out_specs=[pl.BlockSpec((win, 128), lambda i: (i, 0))],
            core_axis_name=("c", "s"),
            dimension_semantics=(pltpu.PARALLEL,),
        )(i_hbm, o_hbm)

    return k(table, idx2)


def _y0_block(q, g, w0a, w0q, w0g, b0):
    qb = q.shape[0]
    wxyz = q[:, 0:3] * q[:, 3:4]
    y0q = _dot(wxyz, w0a) + _dot(q, w0q) + b0
    g2 = g.reshape(qb * K, g.shape[2])
    y0g = _dot(g2, w0g)
    return y0g + jnp.broadcast_to(y0q[:, None, :], (qb, K, C0)).reshape(qb * K, C0)


def _acc_stats(i, s_ref, ss_ref, y):
    @pl.when(i == 0)
    def _():
        s_ref[...] = jnp.zeros_like(s_ref)
        ss_ref[...] = jnp.zeros_like(ss_ref)

    s_ref[...] += jnp.sum(y, axis=0, keepdims=True)
    ss_ref[...] += jnp.sum(y * y, axis=0, keepdims=True)


def _bn_relu(y, s, ss, gamma, beta, count):
    mean = s / count
    var = jnp.maximum(ss / count - mean * mean, 0.0)
    return jnp.maximum((y - mean) * lax.rsqrt(var + 1e-5) * gamma + beta, 0.0)


def _stats0_body(g_ref, q_ref, w0a_ref, w0q_ref, w0g_ref, b0_ref,
                 s_ref, ss_ref):
    y0 = _y0_block(q_ref[...], g_ref[...], w0a_ref[...], w0q_ref[...],
                   w0g_ref[...], b0_ref[...])
    _acc_stats(pl.program_id(0), s_ref, ss_ref, y0)


def _l1_body(g_ref, q_ref, w0a_ref, w0q_ref, w0g_ref, b0_ref,
             s0_ref, ss0_ref, g0_ref, be0_ref, w1_ref, b1_ref,
             y1_ref, gx_ref, s_ref, ss_ref, *, count):
    gx_ref[...] = g_ref[...][:, :, 64:72]
    y0 = _y0_block(q_ref[...], g_ref[...], w0a_ref[...], w0q_ref[...],
                   w0g_ref[...], b0_ref[...])
    x1 = _bn_relu(y0, s0_ref[...], ss0_ref[...], g0_ref[...], be0_ref[...],
                  count)
    y1 = _dot(x1, w1_ref[...]) + b1_ref[...]
    y1_ref[...] = y1.reshape(QB, K, C1)
    _acc_stats(pl.program_id(0), s_ref, ss_ref, y1)


def _l2_body(y1_ref, s1_ref, ss1_ref, g1_ref, be1_ref, w2_ref, b2_ref,
             y2_ref, s_ref, ss_ref, *, count):
    y1 = y1_ref[...].reshape(QB * K, C1)
    x2 = _bn_relu(y1, s1_ref[...], ss1_ref[...], g1_ref[...], be1_ref[...],
                  count)
    y2 = _dot(x2, w2_ref[...]) + b2_ref[...]
    y2_ref[...] = y2.reshape(QB, K, C2)
    _acc_stats(pl.program_id(0), s_ref, ss_ref, y2)


def _agg_body(y2_ref, gx_ref, s2_ref, ss2_ref, g2_ref, be2_ref,
              af_ref, uv_ref, *, count):
    y2 = y2_ref[...].reshape(QB * K, C2)
    feats = _bn_relu(y2, s2_ref[...], ss2_ref[...], g2_ref[...], be2_ref[...],
                     count).reshape(QB, K, C2)
    mx = jnp.max(feats, axis=2)
    e = jnp.exp(mx - jnp.max(mx, axis=1, keepdims=True))
    aw = e / jnp.sum(e, axis=1, keepdims=True)
    af_ref[...] = jnp.sum(aw[:, :, None] * feats, axis=1)
    gx = gx_ref[...][:, :, 0:3]
    uv_ref[...] = jnp.sum(aw[:, :, None] * gx, axis=1)


def _mask(p, q):
    r = lax.broadcasted_iota(jnp.int32, (3, 3), 0)
    c = lax.broadcasted_iota(jnp.int32, (3, 3), 1)
    return ((r == p) & (c == q)).astype(F32)


def _jacobi_svd_r_t(Hm, cs, ct):
    eye = _mask(0, 0) + _mask(1, 1) + _mask(2, 2)
    A = lax.dot_general(Hm, Hm, (((0,), (0,)), ((), ())))
    V = eye
    for _ in range(6):
        for (p, q) in ((0, 1), (0, 2), (1, 2)):
            app = A[p:p + 1, p:p + 1]
            aqq = A[q:q + 1, q:q + 1]
            apq = A[p:p + 1, q:q + 1]
            nz = jnp.abs(apq) > 1e-30
            apq_s = jnp.where(nz, apq, 1.0)
            tau = (aqq - app) / (2.0 * apq_s)
            sg = jnp.where(tau >= 0.0, 1.0, -1.0)
            t = sg / (jnp.abs(tau) + jnp.sqrt(1.0 + tau * tau))
            t = jnp.where(nz, t, 0.0)
            c = lax.rsqrt(1.0 + t * t)
            s = t * c
            J = eye + (c - 1.0) * (_mask(p, p) + _mask(q, q)) \
                + s * _mask(p, q) - s * _mask(q, p)
            A = _dot(lax.dot_general(J, A, (((0,), (0,)), ((), ()))), J)
            V = _dot(V, J)
    l0 = A[0:1, 0:1]
    l1 = A[1:2, 1:2]
    l2 = A[2:3, 2:3]
    detv = jnp.ones_like(l0)
    def swap(li, lj, V, detv, i, j):
        cnd = li < lj
        P = eye - _mask(i, i) - _mask(j, j) + _mask(i, j) + _mask(j, i)
        Vn = jnp.where(cnd, _dot(V, P), V)
        dn = jnp.where(cnd, -detv, detv)
        return jnp.where(cnd, lj, li), jnp.where(cnd, li, lj), Vn, dn
    l0, l1, V, detv = swap(l0, l1, V, detv, 0, 1)
    l0, l2, V, detv = swap(l0, l2, V, detv, 0, 2)
    l1, l2, V, detv = swap(l1, l2, V, detv, 1, 2)
    HV = _dot(Hm, V)
    norm2 = jnp.sum(HV * HV, axis=0, keepdims=True)
    Un = HV * lax.rsqrt(norm2 + 1e-30)
    a = Un[:, 0:1]
    b = Un[:, 1:2]
    u2 = jnp.concatenate(
        [a[1:2] * b[2:3] - a[2:3] * b[1:2],
         a[2:3] * b[0:1] - a[0:1] * b[2:3],
         a[0:1] * b[1:2] - a[1:2] * b[0:1]], axis=0)
    U = jnp.concatenate([a, b, u2], axis=1)
    d_row = jnp.concatenate([jnp.ones_like(detv), jnp.ones_like(detv), detv],
                            axis=1)
    R = lax.dot_general(V * d_row, U, (((1,), (1,)), ((), ())))
    t = ct - lax.dot_general(cs, R, (((1,), (1,)), ((), ())))
    return R, t


def _head_body(af_ref, uv_ref, xyz_ref, z_ref, gt_ref,
               m0w_ref, m0b_ref, m0g_ref, m0be_ref,
               m1w_ref, m1b_ref, m1g_ref, m1be_ref,
               m2w_ref, m2b_ref,
               r_ref, t_ref, w_ref, *, n_rows, n_pts):
    a = af_ref[...]
    y = _dot(a, m0w_ref[...]) + m0b_ref[...]
    s = jnp.sum(y, axis=0, keepdims=True)
    ss = jnp.sum(y * y, axis=0, keepdims=True)
    x = _bn_relu(y, s, ss, m0g_ref[...], m0be_ref[...], float(n_rows))
    y = _dot(x, m1w_ref[...]) + m1b_ref[...]
    s = jnp.sum(y, axis=0, keepdims=True)
    ss = jnp.sum(y * y, axis=0, keepdims=True)
    x = _bn_relu(y, s, ss, m1g_ref[...], m1be_ref[...], float(n_rows))
    w = _dot(x, m2w_ref[...]) + m2b_ref[...]
    w_ref[...] = w.reshape(w_ref.shape)

    for b in range(w_ref.shape[0]):
        gt = gt_ref[b]
        win = jnp.where(gt[:, 1:2] > gt[:, 0:1], 1.0, 0.0)
        wn = win / (jnp.sum(win) + 1e-8)
        src = xyz_ref[b] * z_ref[b]
        uv = uv_ref[pl.ds(b * n_pts, n_pts), 0:2]
        tgt = jnp.concatenate([uv, jnp.ones_like(uv[:, 0:1])], axis=1)
        cs = jnp.sum(wn * src, axis=0, keepdims=True)
        ct = jnp.sum(wn * tgt, axis=0, keepdims=True)
        sc = src - cs
        tc = tgt - ct
        Hm = lax.dot_general(wn * sc, tc, (((0,), (0,)), ((), ())))
        R, t = _jacobi_svd_r_t(Hm, cs, ct)
        r_ref[b] = R
        t_ref[pl.ds(b, 1), :] = t


def kernel(warped_xyz, warped_points, RF3, RF3_index, lidar_z, gt_project,
           c0_W, c0_b, c0_g, c0_be, c1_W, c1_b, c1_g, c1_be,
           c2_W, c2_b, c2_g, c2_be, m0_W, m0_b, m0_g, m0_be,
           m1_W, m1_b, m1_g, m1_be, m2_W, m2_b):
    B, N, _ = warped_xyz.shape
    _, Ci, H, W_ = RF3.shape
    M = H * W_
    BN = B * N
    rows = BN * K
    count = float(rows)

    st = RF3_index.reshape(B, 3, M)
    srows = RF3_index.transpose(0, 2, 3, 1).reshape(B, M, 3)

    feat_t = RF3.transpose(0, 2, 3, 1).reshape(B * M, Ci)
    xyz_t = srows.reshape(B * M, 3)
    table = jnp.concatenate(
        [feat_t, xyz_t, jnp.zeros((B * M, 61), F32)], axis=1)

    idx = _knn(warped_xyz, st, srows)
    g_rows = _sc_gather(table, idx.reshape(rows))
    g3 = g_rows.reshape(BN, K, 128)

    q_cat = jnp.concatenate(
        [warped_xyz.reshape(BN, 3), lidar_z.reshape(BN, 1),
         warped_points.reshape(BN, Ci)], axis=1)

    w0a = c0_W[0:3]
    w0q = jnp.concatenate([jnp.zeros((4, C0), F32), c0_W[6:70]], axis=0)
    w0g = jnp.concatenate([c0_W[70:134], c0_W[3:6],
                           jnp.zeros((61, C0), F32)], axis=0)

    nblk = BN // QB
    g_spec = pl.BlockSpec((QB, K, 128), lambda i: (i, 0, 0))
    q_spec = pl.BlockSpec((QB, 68), lambda i: (i, 0))
    full = lambda *shape: pl.BlockSpec(shape, lambda i: tuple(0 for _ in shape))
    acc_spec = lambda c: pl.BlockSpec((1, c), lambda i: (0, 0))

    s0, ss0 = pl.pallas_call(
        _stats0_body,
        grid=(nblk,),
        in_specs=[g_spec, q_spec, full(3, C0), full(68, C0), full(128, C0),
                  full(1, C0)],
        out_specs=[acc_spec(C0), acc_spec(C0)],
        out_shape=[jax.ShapeDtypeStruct((1, C0), F32)] * 2,
    )(g3, q_cat, w0a, w0q, w0g, c0_b.reshape(1, C0))

    y1, gx3, s1, ss1 = pl.pallas_call(
        functools.partial(_l1_body, count=count),
        grid=(nblk,),
        in_specs=[g_spec, q_spec, full(3, C0), full(68, C0), full(128, C0),
                  full(1, C0), full(1, C0), full(1, C0), full(1, C0),
                  full(1, C0), full(C0, C1), full(1, C1)],
        out_specs=[pl.BlockSpec((QB, K, C1), lambda i: (i, 0, 0)),
                   pl.BlockSpec((QB, K, 8), lambda i: (i, 0, 0)),
                   acc_spec(C1), acc_spec(C1)],
        out_shape=[jax.ShapeDtypeStruct((BN, K, C1), F32),
                   jax.ShapeDtypeStruct((BN, K, 8), F32),
                   jax.ShapeDtypeStruct((1, C1), F32),
                   jax.ShapeDtypeStruct((1, C1), F32)],
    )(g3, q_cat, w0a, w0q, w0g, c0_b.reshape(1, C0), s0, ss0,
      c0_g.reshape(1, C0), c0_be.reshape(1, C0), c1_W, c1_b.reshape(1, C1))

    y2, s2, ss2 = pl.pallas_call(
        functools.partial(_l2_body, count=count),
        grid=(nblk,),
        in_specs=[pl.BlockSpec((QB, K, C1), lambda i: (i, 0, 0)),
                  full(1, C1), full(1, C1), full(1, C1), full(1, C1),
                  full(C1, C2), full(1, C2)],
        out_specs=[pl.BlockSpec((QB, K, C2), lambda i: (i, 0, 0)),
                   acc_spec(C2), acc_spec(C2)],
        out_shape=[jax.ShapeDtypeStruct((BN, K, C2), F32),
                   jax.ShapeDtypeStruct((1, C2), F32),
                   jax.ShapeDtypeStruct((1, C2), F32)],
    )(y1, s1, ss1, c1_g.reshape(1, C1), c1_be.reshape(1, C1),
      c2_W, c2_b.reshape(1, C2))

    af, uv = pl.pallas_call(
        functools.partial(_agg_body, count=count),
        grid=(nblk,),
        in_specs=[pl.BlockSpec((QB, K, C2), lambda i: (i, 0, 0)),
                  pl.BlockSpec((QB, K, 8), lambda i: (i, 0, 0)),
                  full(1, C2), full(1, C2), full(1, C2), full(1, C2)],
        out_specs=[pl.BlockSpec((QB, C2), lambda i: (i, 0)),
                   pl.BlockSpec((QB, 3), lambda i: (i, 0))],
        out_shape=[jax.ShapeDtypeStruct((BN, C2), F32),
                   jax.ShapeDtypeStruct((BN, 3), F32)],
    )(y2, gx3, s2, ss2, c2_g.reshape(1, C2), c2_be.reshape(1, C2))

    C3 = m1_W.shape[1]
    fullb = lambda *shape: pl.BlockSpec(shape, lambda: tuple(0 for _ in shape))
    R, t, weights = pl.pallas_call(
        functools.partial(_head_body, n_rows=BN, n_pts=N),
        in_specs=[fullb(BN, C2), fullb(BN, 3), fullb(B, N, 3), fullb(B, N, 1),
                  fullb(B, N, 2),
                  fullb(C2, C2), fullb(1, C2), fullb(1, C2), fullb(1, C2),
                  fullb(C2, C3), fullb(1, C3), fullb(1, C3), fullb(1, C3),
                  fullb(C3, 2), fullb(1, 2)],
        out_specs=[fullb(B, 3, 3), fullb(B, 3), fullb(B, N, 2)],
        out_shape=[jax.ShapeDtypeStruct((B, 3, 3), F32),
                   jax.ShapeDtypeStruct((B, 3), F32),
                   jax.ShapeDtypeStruct((B, N, 2), F32)],
    )(af, uv, warped_xyz, lidar_z, gt_project,
      m0_W, m0_b.reshape(1, C2), m0_g.reshape(1, C2), m0_be.reshape(1, C2),
      m1_W, m1_b.reshape(1, C3), m1_g.reshape(1, C3), m1_be.reshape(1, C3),
      m2_W, m2_b.reshape(1, 2))

    return (R, t, weights)


# speedup vs baseline: 23.4159x; 1.0166x over previous
import functools

import jax
import jax.numpy as jnp
from jax import lax
from jax.experimental import pallas as pl
from jax.experimental.pallas import tpu as pltpu
from jax.experimental.pallas import tpu_sc as plsc

F32 = jnp.float32
K = 16
QB = 256
C0, C1, C2 = 128, 64, 64


def _dot(a, b):
    return jnp.dot(a, b, preferred_element_type=F32)


def _knn_body(q_ref, st_ref, sr_ref, out_ref):
    q = q_ref[0]
    st = st_ref[0]
    sr = sr_ref[0]
    M = st.shape[1]
    q2 = jnp.sum(q * q, axis=1, keepdims=True)
    s2 = jnp.sum(st * st, axis=0, keepdims=True)
    e = lax.dot_general(q, sr, (((1,), (1,)), ((), ())),
                        preferred_element_type=F32)
    d = (q2 + s2) - 2.0 * e
    base = pl.program_id(0) * M
    NC = M // 128
    DEPTH = 4
    INF = jnp.float32(jnp.inf)
    BIG = jnp.int32(2**30)
    V = [jnp.full(d.shape[:1] + (128,), INF, F32) for _ in range(DEPTH)]
    I = [jnp.zeros(d.shape[:1] + (128,), jnp.int32) for _ in range(DEPTH)]
    for c in range(NC):
        x = d[:, c * 128:(c + 1) * 128]
        xi = jnp.full(x.shape, jnp.int32(c), jnp.int32)
        for t in range(DEPTH):
            lt = x < V[t]
            nv = jnp.minimum(V[t], x)
            x = jnp.maximum(V[t], x)
            ni = jnp.where(lt, xi, I[t])
            xi = jnp.where(lt, I[t], xi)
            V[t] = nv
            I[t] = ni

    lanes = lax.broadcasted_iota(jnp.int32, (d.shape[0], 128), 1)
    picks = []
    for _ in range(K):
        m = jnp.min(V[0], axis=1, keepdims=True)
        head = V[0] <= m
        gsel = jnp.where(head, I[0] * 128 + lanes, BIG)
        g = jnp.min(gsel, axis=1, keepdims=True)
        picks.append(g + base)
        popm = gsel == g
        for i in range(DEPTH - 1):
            V[i] = jnp.where(popm, V[i + 1], V[i])
            I[i] = jnp.where(popm, I[i + 1], I[i])
        V[DEPTH - 1] = jnp.where(popm, INF, V[DEPTH - 1])
    out_ref[0] = jnp.concatenate(picks, axis=1)


def _knn(q_xyz, st, srows):
    B, N, _ = q_xyz.shape
    M = st.shape[2]
    return pl.pallas_call(
        _knn_body,
        grid=(B, N // QB),
        in_specs=[
            pl.BlockSpec((1, QB, 3), lambda b, i: (b, i, 0)),
            pl.BlockSpec((1, 3, M), lambda b, i: (b, 0, 0)),
            pl.BlockSpec((1, M, 3), lambda b, i: (b, 0, 0)),
        ],
        out_specs=pl.BlockSpec((1, QB, K), lambda b, i: (b, i, 0)),
        out_shape=jax.ShapeDtypeStruct((B, N, K), jnp.int32),
    )(q_xyz, st, srows)


def _sc_gather(table, idx_flat):
    n_idx = idx_flat.shape[0]
    idx2 = idx_flat.reshape(1, n_idx)
    mesh = plsc.VectorSubcoreMesh(core_axis_name="c", subcore_axis_name="s")
    win = 128

    @functools.partial(
        pl.kernel,
        out_type=jax.ShapeDtypeStruct((n_idx, 128), F32),
        mesh=mesh,
    )
    def k(t_hbm, i_hbm, o_hbm):
        def body(i_vmem, o_vmem):
            pltpu.sync_copy(t_hbm.at[i_vmem.at[0]], o_vmem)

        pltpu.emit_pipeline(
            body,
            grid=(n_idx // win,),
            in_specs=[pl.BlockSpec((1, win), lambda i: (0, i))],
            out_specs=[pl.BlockSpec((win, 128), lambda i: (i, 0))],
            core_axis_name=("c", "s"),
            dimension_semantics=(pltpu.PARALLEL,),
        )(i_hbm, o_hbm)

    return k(table, idx2)


def _y0_block(q, g, w0a, w0q, w0g, b0):
    qb = q.shape[0]
    wxyz = q[:, 0:3] * q[:, 3:4]
    y0q = _dot(wxyz, w0a) + _dot(q, w0q) + b0
    g2 = g.reshape(qb * K, g.shape[2])
    y0g = _dot(g2, w0g)
    return y0g + jnp.broadcast_to(y0q[:, None, :], (qb, K, C0)).reshape(qb * K, C0)


def _acc_stats(i, s_ref, ss_ref, y):
    @pl.when(i == 0)
    def _():
        s_ref[...] = jnp.zeros_like(s_ref)
        ss_ref[...] = jnp.zeros_like(ss_ref)

    s_ref[...] += jnp.sum(y, axis=0, keepdims=True)
    ss_ref[...] += jnp.sum(y * y, axis=0, keepdims=True)


def _bn_relu(y, s, ss, gamma, beta, count):
    mean = s / count
    var = jnp.maximum(ss / count - mean * mean, 0.0)
    return jnp.maximum((y - mean) * lax.rsqrt(var + 1e-5) * gamma + beta, 0.0)


def _stats0_body(g_ref, q_ref, w0a_ref, w0q_ref, w0g_ref, b0_ref,
                 s_ref, ss_ref):
    y0 = _y0_block(q_ref[...], g_ref[...], w0a_ref[...], w0q_ref[...],
                   w0g_ref[...], b0_ref[...])
    _acc_stats(pl.program_id(0), s_ref, ss_ref, y0)


def _l1_body(g_ref, q_ref, w0a_ref, w0q_ref, w0g_ref, b0_ref,
             s0_ref, ss0_ref, g0_ref, be0_ref, w1_ref, b1_ref,
             y1_ref, gx_ref, s_ref, ss_ref, *, count):
    gx_ref[...] = g_ref[...][:, :, 64:72]
    y0 = _y0_block(q_ref[...], g_ref[...], w0a_ref[...], w0q_ref[...],
                   w0g_ref[...], b0_ref[...])
    x1 = _bn_relu(y0, s0_ref[...], ss0_ref[...], g0_ref[...], be0_ref[...],
                  count)
    y1 = _dot(x1, w1_ref[...]) + b1_ref[...]
    y1_ref[...] = y1.reshape(QB, K, C1)
    _acc_stats(pl.program_id(0), s_ref, ss_ref, y1)


def _l2_body(y1_ref, s1_ref, ss1_ref, g1_ref, be1_ref, w2_ref, b2_ref,
             y2_ref, s_ref, ss_ref, *, count):
    y1 = y1_ref[...].reshape(QB * K, C1)
    x2 = _bn_relu(y1, s1_ref[...], ss1_ref[...], g1_ref[...], be1_ref[...],
                  count)
    y2 = _dot(x2, w2_ref[...]) + b2_ref[...]
    y2_ref[...] = y2.reshape(QB, K, C2)
    _acc_stats(pl.program_id(0), s_ref, ss_ref, y2)


def _agg_body(y2_ref, gx_ref, s2_ref, ss2_ref, g2_ref, be2_ref,
              af_ref, uv_ref, *, count):
    y2 = y2_ref[...].reshape(QB * K, C2)
    feats = _bn_relu(y2, s2_ref[...], ss2_ref[...], g2_ref[...], be2_ref[...],
                     count).reshape(QB, K, C2)
    mx = jnp.max(feats, axis=2)
    e = jnp.exp(mx - jnp.max(mx, axis=1, keepdims=True))
    aw = e / jnp.sum(e, axis=1, keepdims=True)
    af_ref[...] = jnp.sum(aw[:, :, None] * feats, axis=1)
    gx = gx_ref[...][:, :, 0:3]
    uv_ref[...] = jnp.sum(aw[:, :, None] * gx, axis=1)


def _mask(p, q):
    r = lax.broadcasted_iota(jnp.int32, (3, 3), 0)
    c = lax.broadcasted_iota(jnp.int32, (3, 3), 1)
    return ((r == p) & (c == q)).astype(F32)


def _jacobi_svd_r_t(Hm, cs, ct):
    eye = _mask(0, 0) + _mask(1, 1) + _mask(2, 2)
    rowi = lax.broadcasted_iota(jnp.int32, (3, 3), 0)
    coli = lax.broadcasted_iota(jnp.int32, (3, 3), 1)
    A = lax.dot_general(Hm, Hm, (((0,), (0,)), ((), ())))
    V = eye
    for _ in range(4):
        for (p, q) in ((0, 1), (0, 2), (1, 2)):
            app = A[p:p + 1, p:p + 1]
            aqq = A[q:q + 1, q:q + 1]
            apq = A[p:p + 1, q:q + 1]
            nz = jnp.abs(apq) > 1e-30
            apq_s = jnp.where(nz, apq, 1.0)
            tau = (aqq - app) / (2.0 * apq_s)
            sg = jnp.where(tau >= 0.0, 1.0, -1.0)
            t = sg / (jnp.abs(tau) + jnp.sqrt(1.0 + tau * tau))
            t = jnp.where(nz, t, 0.0)
            c = lax.rsqrt(1.0 + t * t)
            s = t * c
            ap_r = A[p:p + 1, :]
            aq_r = A[q:q + 1, :]
            A = jnp.where(rowi == p, c * ap_r - s * aq_r,
                          jnp.where(rowi == q, s * ap_r + c * aq_r, A))
            ap_c = A[:, p:p + 1]
            aq_c = A[:, q:q + 1]
            A = jnp.where(coli == p, c * ap_c - s * aq_c,
                          jnp.where(coli == q, s * ap_c + c * aq_c, A))
            vp = V[:, p:p + 1]
            vq = V[:, q:q + 1]
            V = jnp.where(coli == p, c * vp - s * vq,
                          jnp.where(coli == q, s * vp + c * vq, V))
    l0 = A[0:1, 0:1]
    l1 = A[1:2, 1:2]
    l2 = A[2:3, 2:3]
    detv = jnp.ones_like(l0)
    def swap(li, lj, V, detv, i, j):
        cnd = li < lj
        Vsw = jnp.where(coli == i, V[:, j:j + 1],
                        jnp.where(coli == j, V[:, i:i + 1], V))
        Vn = jnp.where(cnd, Vsw, V)
        dn = jnp.where(cnd, -detv, detv)
        return jnp.where(cnd, lj, li), jnp.where(cnd, li, lj), Vn, dn
    l0, l1, V, detv = swap(l0, l1, V, detv, 0, 1)
    l0, l2, V, detv = swap(l0, l2, V, detv, 0, 2)
    l1, l2, V, detv = swap(l1, l2, V, detv, 1, 2)
    HV = _dot(Hm, V)
    norm2 = jnp.sum(HV * HV, axis=0, keepdims=True)
    Un = HV * lax.rsqrt(norm2 + 1e-30)
    a = Un[:, 0:1]
    b = Un[:, 1:2]
    u2 = jnp.concatenate(
        [a[1:2] * b[2:3] - a[2:3] * b[1:2],
         a[2:3] * b[0:1] - a[0:1] * b[2:3],
         a[0:1] * b[1:2] - a[1:2] * b[0:1]], axis=0)
    U = jnp.concatenate([a, b, u2], axis=1)
    d_row = jnp.concatenate([jnp.ones_like(detv), jnp.ones_like(detv), detv],
                            axis=1)
    R = lax.dot_general(V * d_row, U, (((1,), (1,)), ((), ())))
    t = ct - lax.dot_general(cs, R, (((1,), (1,)), ((), ())))
    return R, t


def _head_body(af_ref, uv_ref, xyz_ref, z_ref, gt_ref,
               m0w_ref, m0b_ref, m0g_ref, m0be_ref,
               m1w_ref, m1b_ref, m1g_ref, m1be_ref,
               m2w_ref, m2b_ref,
               r_ref, t_ref, w_ref, *, n_rows, n_pts):
    a = af_ref[...]
    y = _dot(a, m0w_ref[...]) + m0b_ref[...]
    s = jnp.sum(y, axis=0, keepdims=True)
    ss = jnp.sum(y * y, axis=0, keepdims=True)
    x = _bn_relu(y, s, ss, m0g_ref[...], m0be_ref[...], float(n_rows))
    y = _dot(x, m1w_ref[...]) + m1b_ref[...]
    s = jnp.sum(y, axis=0, keepdims=True)
    ss = jnp.sum(y * y, axis=0, keepdims=True)
    x = _bn_relu(y, s, ss, m1g_ref[...], m1be_ref[...], float(n_rows))
    w = _dot(x, m2w_ref[...]) + m2b_ref[...]
    w_ref[...] = w.reshape(w_ref.shape)

    for b in range(w_ref.shape[0]):
        gt = gt_ref[b]
        win = jnp.where(gt[:, 1:2] > gt[:, 0:1], 1.0, 0.0)
        wn = win / (jnp.sum(win) + 1e-8)
        src = xyz_ref[b] * z_ref[b]
        uv = uv_ref[pl.ds(b * n_pts, n_pts), 0:2]
        tgt = jnp.concatenate([uv, jnp.ones_like(uv[:, 0:1])], axis=1)
        cs = jnp.sum(wn * src, axis=0, keepdims=True)
        ct = jnp.sum(wn * tgt, axis=0, keepdims=True)
        sc = src - cs
        tc = tgt - ct
        Hm = lax.dot_general(wn * sc, tc, (((0,), (0,)), ((), ())))
        R, t = _jacobi_svd_r_t(Hm, cs, ct)
        r_ref[b] = R
        t_ref[pl.ds(b, 1), :] = t


def kernel(warped_xyz, warped_points, RF3, RF3_index, lidar_z, gt_project,
           c0_W, c0_b, c0_g, c0_be, c1_W, c1_b, c1_g, c1_be,
           c2_W, c2_b, c2_g, c2_be, m0_W, m0_b, m0_g, m0_be,
           m1_W, m1_b, m1_g, m1_be, m2_W, m2_b):
    B, N, _ = warped_xyz.shape
    _, Ci, H, W_ = RF3.shape
    M = H * W_
    BN = B * N
    rows = BN * K
    count = float(rows)

    st = RF3_index.reshape(B, 3, M)
    srows = RF3_index.transpose(0, 2, 3, 1).reshape(B, M, 3)

    feat_t = RF3.transpose(0, 2, 3, 1).reshape(B * M, Ci)
    xyz_t = srows.reshape(B * M, 3)
    table = jnp.concatenate(
        [feat_t, xyz_t, jnp.zeros((B * M, 61), F32)], axis=1)

    idx = _knn(warped_xyz, st, srows)
    g_rows = _sc_gather(table, idx.reshape(rows))
    g3 = g_rows.reshape(BN, K, 128)

    q_cat = jnp.concatenate(
        [warped_xyz.reshape(BN, 3), lidar_z.reshape(BN, 1),
         warped_points.reshape(BN, Ci)], axis=1)

    w0a = c0_W[0:3]
    w0q = jnp.concatenate([jnp.zeros((4, C0), F32), c0_W[6:70]], axis=0)
    w0g = jnp.concatenate([c0_W[70:134], c0_W[3:6],
                           jnp.zeros((61, C0), F32)], axis=0)

    nblk = BN // QB
    g_spec = pl.BlockSpec((QB, K, 128), lambda i: (i, 0, 0))
    q_spec = pl.BlockSpec((QB, 68), lambda i: (i, 0))
    full = lambda *shape: pl.BlockSpec(shape, lambda i: tuple(0 for _ in shape))
    acc_spec = lambda c: pl.BlockSpec((1, c), lambda i: (0, 0))

    s0, ss0 = pl.pallas_call(
        _stats0_body,
        grid=(nblk,),
        in_specs=[g_spec, q_spec, full(3, C0), full(68, C0), full(128, C0),
                  full(1, C0)],
        out_specs=[acc_spec(C0), acc_spec(C0)],
        out_shape=[jax.ShapeDtypeStruct((1, C0), F32)] * 2,
    )(g3, q_cat, w0a, w0q, w0g, c0_b.reshape(1, C0))

    y1, gx3, s1, ss1 = pl.pallas_call(
        functools.partial(_l1_body, count=count),
        grid=(nblk,),
        in_specs=[g_spec, q_spec, full(3, C0), full(68, C0), full(128, C0),
                  full(1, C0), full(1, C0), full(1, C0), full(1, C0),
                  full(1, C0), full(C0, C1), full(1, C1)],
        out_specs=[pl.BlockSpec((QB, K, C1), lambda i: (i, 0, 0)),
                   pl.BlockSpec((QB, K, 8), lambda i: (i, 0, 0)),
                   acc_spec(C1), acc_spec(C1)],
        out_shape=[jax.ShapeDtypeStruct((BN, K, C1), F32),
                   jax.ShapeDtypeStruct((BN, K, 8), F32),
                   jax.ShapeDtypeStruct((1, C1), F32),
                   jax.ShapeDtypeStruct((1, C1), F32)],
    )(g3, q_cat, w0a, w0q, w0g, c0_b.reshape(1, C0), s0, ss0,
      c0_g.reshape(1, C0), c0_be.reshape(1, C0), c1_W, c1_b.reshape(1, C1))

    y2, s2, ss2 = pl.pallas_call(
        functools.partial(_l2_body, count=count),
        grid=(nblk,),
        in_specs=[pl.BlockSpec((QB, K, C1), lambda i: (i, 0, 0)),
                  full(1, C1), full(1, C1), full(1, C1), full(1, C1),
                  full(C1, C2), full(1, C2)],
        out_specs=[pl.BlockSpec((QB, K, C2), lambda i: (i, 0, 0)),
                   acc_spec(C2), acc_spec(C2)],
        out_shape=[jax.ShapeDtypeStruct((BN, K, C2), F32),
                   jax.ShapeDtypeStruct((1, C2), F32),
                   jax.ShapeDtypeStruct((1, C2), F32)],
    )(y1, s1, ss1, c1_g.reshape(1, C1), c1_be.reshape(1, C1),
      c2_W, c2_b.reshape(1, C2))

    af, uv = pl.pallas_call(
        functools.partial(_agg_body, count=count),
        grid=(nblk,),
        in_specs=[pl.BlockSpec((QB, K, C2), lambda i: (i, 0, 0)),
                  pl.BlockSpec((QB, K, 8), lambda i: (i, 0, 0)),
                  full(1, C2), full(1, C2), full(1, C2), full(1, C2)],
        out_specs=[pl.BlockSpec((QB, C2), lambda i: (i, 0)),
                   pl.BlockSpec((QB, 3), lambda i: (i, 0))],
        out_shape=[jax.ShapeDtypeStruct((BN, C2), F32),
                   jax.ShapeDtypeStruct((BN, 3), F32)],
    )(y2, gx3, s2, ss2, c2_g.reshape(1, C2), c2_be.reshape(1, C2))

    C3 = m1_W.shape[1]
    fullb = lambda *shape: pl.BlockSpec(shape, lambda: tuple(0 for _ in shape))
    R, t, weights = pl.pallas_call(
        functools.partial(_head_body, n_rows=BN, n_pts=N),
        in_specs=[fullb(BN, C2), fullb(BN, 3), fullb(B, N, 3), fullb(B, N, 1),
                  fullb(B, N, 2),
                  fullb(C2, C2), fullb(1, C2), fullb(1, C2), fullb(1, C2),
                  fullb(C2, C3), fullb(1, C3), fullb(1, C3), fullb(1, C3),
                  fullb(C3, 2), fullb(1, 2)],
        out_specs=[fullb(B, 3, 3), fullb(B, 3), fullb(B, N, 2)],
        out_shape=[jax.ShapeDtypeStruct((B, 3, 3), F32),
                   jax.ShapeDtypeStruct((B, 3), F32),
                   jax.ShapeDtypeStruct((B, N, 2), F32)],
    )(af, uv, warped_xyz, lidar_z, gt_project,
      m0_W, m0_b.reshape(1, C2), m0_g.reshape(1, C2), m0_be.reshape(1, C2),
      m1_W, m1_b.reshape(1, C3), m1_g.reshape(1, C3), m1_be.reshape(1, C3),
      m2_W, m2_b.reshape(1, 2))

    return (R, t, weights)
